# baseline (speedup 1.0000x reference)
import dataclasses
import functools

import jax
import jax.numpy as jnp
from jax.experimental import pallas as pl
from jax.experimental.pallas import tpu as pltpu
from jax.experimental.pallas import tpu_sc as plsc

_B = 16384
_K = 1024
_R = 16


def _sc_permute(x, perm):
    mesh = plsc.VectorSubcoreMesh(core_axis_name="c", subcore_axis_name="s")
    cp = pltpu.CompilerParams()
    if "needs_layout_passes" in pltpu.CompilerParams.__dataclass_fields__:
        cp = dataclasses.replace(cp, needs_layout_passes=False)

    @functools.partial(
        pl.kernel,
        mesh=mesh,
        out_type=jax.ShapeDtypeStruct((_B, _K), jnp.float32),
        scratch_types=[pltpu.VMEM((_K,), jnp.int32)],
        compiler_params=cp,
    )
    def run(x_hbm, perm_hbm, out_hbm, idx_v):
        pltpu.sync_copy(perm_hbm, idx_v)

        def body(in_v, out_v):
            @pl.loop(0, _K // 16)
            def _(kc):
                col = idx_v[pl.ds(kc * 16, 16)]

                @pl.loop(0, _R)
                def _(r):
                    row = jnp.full((16,), r, jnp.int32)
                    out_v[r, pl.ds(kc * 16, 16)] = plsc.load_gather(
                        in_v, [row, col]
                    )

        pltpu.emit_pipeline(
            body,
            grid=(_B // _R,),
            in_specs=[pl.BlockSpec((_R, _K), lambda i: (i, 0))],
            out_specs=[pl.BlockSpec((_R, _K), lambda i: (i, 0))],
            core_axis_name=("c", "s"),
            dimension_semantics=(pltpu.PARALLEL,),
        )(x_hbm, out_hbm)

    return run(x, perm)


def kernel(x, matrix):
    perm = jnp.transpose(matrix).reshape(_K).astype(jnp.int32)
    out = _sc_permute(x, perm)
    return out.reshape(_B, 32, 32)


# speedup vs baseline: 1.3231x; 1.3231x over previous
import dataclasses
import functools

import jax
import jax.numpy as jnp
from jax.experimental import pallas as pl
from jax.experimental.pallas import tpu as pltpu
from jax.experimental.pallas import tpu_sc as plsc

_B = 16384
_K = 1024
_R = 16


def _sc_permute(x, perm):
    mesh = plsc.VectorSubcoreMesh(core_axis_name="c", subcore_axis_name="s")
    cp = pltpu.CompilerParams()
    if "needs_layout_passes" in pltpu.CompilerParams.__dataclass_fields__:
        cp = dataclasses.replace(cp, needs_layout_passes=False)

    @functools.partial(
        pl.kernel,
        mesh=mesh,
        out_type=jax.ShapeDtypeStruct((_B, _K), jnp.float32),
        scratch_types=[pltpu.VMEM((_K,), jnp.int32)],
        compiler_params=cp,
    )
    def run(x_hbm, perm_hbm, out_hbm, idx_v):
        pltpu.sync_copy(perm_hbm, idx_v)

        def body(in_v, out_v):
            @pl.loop(0, _K // 16)
            def _(kc):
                col = idx_v[pl.ds(kc * 16, 16)]

                @plsc.parallel_loop(0, _R, 1, unroll=8)
                def _(r):
                    row = jnp.full((16,), r, jnp.int32)
                    out_v[r, pl.ds(kc * 16, 16)] = plsc.load_gather(
                        in_v, [row, col]
                    )

        pltpu.emit_pipeline(
            body,
            grid=(_B // _R,),
            in_specs=[pl.BlockSpec((_R, _K), lambda i: (i, 0))],
            out_specs=[pl.BlockSpec((_R, _K), lambda i: (i, 0))],
            core_axis_name=("c", "s"),
            dimension_semantics=(pltpu.PARALLEL,),
        )(x_hbm, out_hbm)

    return run(x, perm)


def kernel(x, matrix):
    perm = jnp.transpose(matrix).reshape(_K).astype(jnp.int32)
    out = _sc_permute(x, perm)
    return out.reshape(_B, 32, 32)
